# baseline (speedup 1.0000x reference)
import functools

import jax
import jax.numpy as jnp
from jax import lax
from jax.experimental import pallas as pl
from jax.experimental.pallas import tpu as pltpu
from jax.experimental.pallas import tpu_sc as plsc

NC = 2
NS = 16
NW = NC * NS
LANES = 16
CHUNK = 128
PAD_ROWS = 240


def _sc_mesh():
    return plsc.VectorSubcoreMesh(
        core_axis_name="c", subcore_axis_name="s", num_cores=NC, num_subcores=NS
    )


_SC_PARAMS = pltpu.CompilerParams(use_tc_tiling_on_sc=False)


def _zeros16():
    return jnp.zeros((LANES,), jnp.float32)


def _make_deg_kernel(n_chunks, rows_pad, stripe):
    @functools.partial(
        pl.kernel,
        mesh=_sc_mesh(),
        out_type=jax.ShapeDtypeStruct((NC, rows_pad), jnp.float32),
        compiler_params=_SC_PARAMS,
        scratch_types=[
            pltpu.VMEM_SHARED((rows_pad,), jnp.float32),
            pltpu.VMEM((n_chunks, CHUNK), jnp.int32),
            pltpu.VMEM((CHUNK,), jnp.float32),
            pltpu.VMEM((stripe,), jnp.float32),
        ],
    )
    def deg_kernel(dst_hbm, degp_hbm, deg_sp, idx_v, ones_v, zbuf_v):
        c = lax.axis_index("c")
        s = lax.axis_index("s")
        w = c * NS + s
        for i in range(stripe // LANES):
            zbuf_v[pl.ds(i * LANES, LANES)] = _zeros16()
        for i in range(CHUNK // LANES):
            ones_v[pl.ds(i * LANES, LANES)] = jnp.full((LANES,), 1.0, jnp.float32)
        pltpu.sync_copy(zbuf_v, deg_sp.at[pl.ds(s * stripe, stripe)])
        pltpu.sync_copy(dst_hbm.at[w], idx_v)
        plsc.subcore_barrier()

        def body(j, _):
            pltpu.sync_copy(ones_v, deg_sp.at[idx_v.at[j]], add=True)
            return _

        lax.fori_loop(0, n_chunks, body, None)
        plsc.subcore_barrier()
        pltpu.sync_copy(
            deg_sp.at[pl.ds(s * stripe, stripe)],
            degp_hbm.at[c, pl.ds(s * stripe, stripe)],
        )

    return deg_kernel


def _make_agg_kernel(n_chunks, rows_pad, stripe, out_ch):
    nbuf = 8
    n_groups = n_chunks // nbuf

    @functools.partial(
        pl.kernel,
        mesh=_sc_mesh(),
        out_type=jax.ShapeDtypeStruct((NC, rows_pad, out_ch), jnp.float32),
        compiler_params=_SC_PARAMS,
        scratch_types=[
            pltpu.VMEM_SHARED((rows_pad, out_ch), jnp.float32),
            pltpu.VMEM((n_chunks, CHUNK), jnp.int32),
            pltpu.VMEM((n_chunks, CHUNK), jnp.int32),
            pltpu.VMEM((nbuf, CHUNK, out_ch), jnp.float32),
            pltpu.VMEM((LANES, out_ch), jnp.float32),
            pltpu.SemaphoreType.DMA,
            pltpu.SemaphoreType.DMA,
        ],
    )
    def agg_kernel(src_hbm, dst_hbm, hs_hbm, agg_hbm,
                   agg_sp, src_v, dst_v, rows_v, zbuf, gsem, ssem):
        c = lax.axis_index("c")
        s = lax.axis_index("s")
        w = c * NS + s
        for r in range(LANES):
            for q in range(out_ch // LANES):
                zbuf[r, pl.ds(q * LANES, LANES)] = _zeros16()

        def zbody(k, _):
            pltpu.sync_copy(
                zbuf, agg_sp.at[pl.ds(s * stripe + k * LANES, LANES), :]
            )
            return _

        lax.fori_loop(0, stripe // LANES, zbody, None)
        pltpu.sync_copy(src_hbm.at[w], src_v)
        pltpu.sync_copy(dst_hbm.at[w], dst_v)
        plsc.subcore_barrier()

        for t in range(nbuf - 1):
            pltpu.async_copy(hs_hbm.at[src_v.at[t]], rows_v.at[t], gsem)

        def group(jj, _):
            for t in range(nbuf):
                j = jj * nbuf + t

                @pl.when(j > 0)
                def _drain():
                    tp = (t - 1) % nbuf
                    pltpu.make_async_copy(
                        rows_v.at[tp], agg_sp.at[dst_v.at[j - 1]], ssem
                    ).wait()

                @pl.when(j + nbuf - 1 < n_chunks)
                def _prefetch():
                    tn = (t + nbuf - 1) % nbuf
                    pltpu.async_copy(
                        hs_hbm.at[src_v.at[j + nbuf - 1]], rows_v.at[tn], gsem
                    )

                pltpu.make_async_copy(
                    hs_hbm.at[src_v.at[j]], rows_v.at[t], gsem
                ).wait()
                pltpu.async_copy(
                    rows_v.at[t], agg_sp.at[dst_v.at[j]], ssem, add=True
                )
            return _

        lax.fori_loop(0, n_groups, group, None)
        pltpu.make_async_copy(
            rows_v.at[nbuf - 1], agg_sp.at[dst_v.at[n_chunks - 1]], ssem
        ).wait()
        plsc.subcore_barrier()
        pltpu.sync_copy(
            agg_sp.at[pl.ds(s * stripe, stripe), :],
            agg_hbm.at[c, pl.ds(s * stripe, stripe), :],
        )

    return agg_kernel


def _dinv_cols(degp_ref, i, pk):
    sl = pl.ds(i * pk, pk)
    deg = degp_ref[0, sl, :] + degp_ref[1, sl, :] + 1.0
    dinv_t = lax.rsqrt(deg).T
    da = jnp.concatenate([dinv_t[:, 0:1], dinv_t[:, 1:2]], axis=0)
    db = jnp.concatenate([dinv_t[:, 2:3], dinv_t[:, 3:4]], axis=0)
    return da, db


def _encode_block(pk, x_ref, w_ref, degp_ref, hs_ref):
    i = pl.program_id(0)
    da, db = _dinv_cols(degp_ref, i, pk)
    w = w_ref[...]
    ha = jnp.dot(x_ref[0:256, :] * da, w, preferred_element_type=jnp.float32)
    hb = jnp.dot(x_ref[256:512, :] * db, w, preferred_element_type=jnp.float32)
    hs_ref[...] = jnp.concatenate([ha, hb], axis=1)


def _decode_block(pk, aggp_ref, hs_ref, degp_ref, b_ref, wd_ref, out_ref):
    i = pl.program_id(0)
    da, db = _dinv_cols(degp_ref, i, pk)
    s128 = aggp_ref[0] + aggp_ref[1] + hs_ref[...]
    b = b_ref[...]
    wd = wd_ref[...]

    def half(sv, dv):
        z = jnp.maximum(sv * dv + b, 0.0)
        lg = lax.dot_general(z, wd, (((1,), (1,)), ((), ())),
                             preferred_element_type=jnp.float32)
        m = jnp.max(lg, axis=1, keepdims=True)
        e = jnp.exp(lg - m)
        return e / jnp.sum(e, axis=1, keepdims=True)

    oa = half(s128[:, 0:64], da)
    ob = half(s128[:, 64:128], db)
    out_ref[...] = jnp.concatenate([oa, ob], axis=0)


def kernel(x, edge_index, W_enc, b_enc, W_dec):
    n, in_ch = x.shape
    out_ch = W_enc.shape[1]
    e = edge_index.shape[1]

    src = edge_index[0].astype(jnp.int32)
    dst = edge_index[1].astype(jnp.int32)
    n_chunks = -(-e // (NW * CHUNK))
    n_chunks = -(-n_chunks // 8) * 8
    e_pad = NW * n_chunks * CHUNK
    rows_pad = -(-(n + PAD_ROWS) // 512) * 512
    stripe = rows_pad // NS
    npad = e_pad - e
    pad_src = jnp.arange(npad, dtype=jnp.int32) % n
    pad_dst = n + jnp.arange(npad, dtype=jnp.int32) % PAD_ROWS
    src_p = jnp.concatenate([src, pad_src])
    dst_p = jnp.concatenate([dst, pad_dst])

    def gmap(v):
        u = v % 512
        p = u // 256
        return (v - u) + 2 * (u - 256 * p) + p

    src3 = gmap(src_p).reshape(NW, n_chunks, CHUNK)
    dst3 = gmap(dst_p).reshape(NW, n_chunks, CHUNK)
    dst3n = dst_p.reshape(NW, n_chunks, CHUNK)

    degp = _make_deg_kernel(n_chunks, rows_pad, stripe)(dst3n)
    degp3 = degp.reshape(NC, rows_pad // 128, 128)

    blk = 512
    grid = rows_pad // blk
    pk = blk // 128
    pk2 = blk * out_ch // 128
    hs128 = pl.pallas_call(
        functools.partial(_encode_block, pk),
        grid=(grid,),
        in_specs=[
            pl.BlockSpec((blk, in_ch), lambda i: (i, 0)),
            pl.BlockSpec((in_ch, out_ch), lambda i: (0, 0)),
            pl.BlockSpec((NC, rows_pad // 128, 128), lambda i: (0, 0, 0)),
        ],
        out_specs=pl.BlockSpec((pk2, 128), lambda i: (i, 0)),
        out_shape=jax.ShapeDtypeStruct((rows_pad * out_ch // 128, 128),
                                       jnp.float32),
    )(x, W_enc, degp3)
    hs = hs128.reshape(rows_pad, out_ch)

    aggp = _make_agg_kernel(n_chunks, rows_pad, stripe, out_ch)(src3, dst3, hs)
    aggp128 = aggp.reshape(NC, rows_pad * out_ch // 128, 128)

    out = pl.pallas_call(
        functools.partial(_decode_block, pk),
        grid=(grid,),
        in_specs=[
            pl.BlockSpec((NC, pk2, 128), lambda i: (0, i, 0)),
            pl.BlockSpec((pk2, 128), lambda i: (i, 0)),
            pl.BlockSpec((NC, rows_pad // 128, 128), lambda i: (0, 0, 0)),
            pl.BlockSpec((1, out_ch), lambda i: (0, 0)),
            pl.BlockSpec((in_ch, out_ch), lambda i: (0, 0)),
        ],
        out_specs=pl.BlockSpec((blk, in_ch), lambda i: (i, 0)),
        out_shape=jax.ShapeDtypeStruct((n, in_ch), jnp.float32),
    )(aggp128, hs128, degp3, b_enc.reshape(1, out_ch), W_dec)

    return out


# speedup vs baseline: 1.2507x; 1.2507x over previous
import functools

import jax
import jax.numpy as jnp
from jax import lax
from jax.experimental import pallas as pl
from jax.experimental.pallas import tpu as pltpu
from jax.experimental.pallas import tpu_sc as plsc

NC = 2
NS = 16
NW = NC * NS
LANES = 16
CHUNK = 128
BLK = 1024


def _sc_mesh():
    return plsc.VectorSubcoreMesh(
        core_axis_name="c", subcore_axis_name="s", num_cores=NC, num_subcores=NS
    )


_SC_PARAMS = pltpu.CompilerParams(use_tc_tiling_on_sc=False)


def _zeros16():
    return jnp.zeros((LANES,), jnp.float32)


def _gmap16(v):
    u = lax.bitwise_and(v, BLK - 1)
    p = lax.shift_right_logical(u, (BLK // 2).bit_length() - 1)
    r = lax.bitwise_and(u, BLK // 2 - 1)
    return (v - u) + 2 * r + p


def _make_deg_kernel(n_chunks, n_full, rem_chunks, rows_pad, stripe):
    epw = n_chunks * CHUNK

    @functools.partial(
        pl.kernel,
        mesh=_sc_mesh(),
        out_type=jax.ShapeDtypeStruct((NC, rows_pad), jnp.float32),
        compiler_params=_SC_PARAMS,
        scratch_types=[
            pltpu.VMEM_SHARED((rows_pad,), jnp.float32),
            pltpu.VMEM((n_chunks, CHUNK), jnp.int32),
            pltpu.VMEM((CHUNK,), jnp.float32),
            pltpu.VMEM((stripe,), jnp.float32),
        ],
    )
    def deg_kernel(ei_hbm, degp_hbm, deg_sp, idx_v, ones_v, zbuf_v):
        c = lax.axis_index("c")
        s = lax.axis_index("s")
        w = c * NS + s
        for i in range(stripe // LANES):
            zbuf_v[pl.ds(i * LANES, LANES)] = _zeros16()
        for i in range(CHUNK // LANES):
            ones_v[pl.ds(i * LANES, LANES)] = jnp.full((LANES,), 1.0, jnp.float32)
        pltpu.sync_copy(zbuf_v, deg_sp.at[pl.ds(s * stripe, stripe)])

        @pl.when(w < n_full)
        def _full():
            pltpu.sync_copy(ei_hbm.at[1, pl.ds(w * n_chunks, n_chunks), :],
                            idx_v)

        if rem_chunks:
            @pl.when(w == n_full)
            def _rem():
                pltpu.sync_copy(
                    ei_hbm.at[1, pl.ds(w * n_chunks, rem_chunks), :],
                    idx_v.at[pl.ds(0, rem_chunks), :])

        n_my = jnp.where(w < n_full, n_chunks,
                         jnp.where(w == n_full, rem_chunks, 0))
        plsc.subcore_barrier()

        def body(j, _):
            pltpu.sync_copy(ones_v, deg_sp.at[idx_v.at[j]], add=True)
            return _

        lax.fori_loop(0, n_my, body, None)
        plsc.subcore_barrier()
        pltpu.sync_copy(
            deg_sp.at[pl.ds(s * stripe, stripe)],
            degp_hbm.at[c, pl.ds(s * stripe, stripe)],
        )

    return deg_kernel


def _make_agg_kernel(n_chunks, n_full, rem_chunks, rows_pad, stripe, out_ch):
    epw = n_chunks * CHUNK
    nbuf = 4

    @functools.partial(
        pl.kernel,
        mesh=_sc_mesh(),
        out_type=jax.ShapeDtypeStruct((NC, rows_pad, out_ch), jnp.float32),
        compiler_params=_SC_PARAMS,
        scratch_types=[
            pltpu.VMEM_SHARED((rows_pad, out_ch), jnp.float32),
            pltpu.VMEM((n_chunks, CHUNK), jnp.int32),
            pltpu.VMEM((n_chunks, CHUNK), jnp.int32),
            pltpu.VMEM((nbuf, CHUNK), jnp.int32),
            pltpu.VMEM((nbuf, CHUNK), jnp.int32),
            pltpu.VMEM((nbuf, CHUNK, out_ch), jnp.float32),
            pltpu.VMEM((LANES, out_ch), jnp.float32),
            pltpu.SemaphoreType.DMA,
            pltpu.SemaphoreType.DMA,
        ],
    )
    def agg_kernel(ei_hbm, hs_hbm, agg_hbm,
                   agg_sp, src_v, dst_v, gidx_v, sidx_v, rows_v, zbuf,
                   gsem, ssem):
        c = lax.axis_index("c")
        s = lax.axis_index("s")
        w = c * NS + s
        for r in range(LANES):
            for q in range(out_ch // LANES):
                zbuf[r, pl.ds(q * LANES, LANES)] = _zeros16()

        def zbody(k, _):
            pltpu.sync_copy(
                zbuf, agg_sp.at[pl.ds(s * stripe + k * LANES, LANES), :]
            )
            return _

        lax.fori_loop(0, stripe // LANES, zbody, None)

        @pl.when(w < n_full)
        def _full():
            pltpu.sync_copy(ei_hbm.at[0, pl.ds(w * n_chunks, n_chunks), :],
                            src_v)
            pltpu.sync_copy(ei_hbm.at[1, pl.ds(w * n_chunks, n_chunks), :],
                            dst_v)

        if rem_chunks:
            @pl.when(w == n_full)
            def _rem():
                pltpu.sync_copy(ei_hbm.at[0, pl.ds(w * n_chunks, rem_chunks), :],
                                src_v.at[pl.ds(0, rem_chunks), :])
                pltpu.sync_copy(ei_hbm.at[1, pl.ds(w * n_chunks, rem_chunks), :],
                                dst_v.at[pl.ds(0, rem_chunks), :])

        n_my = jnp.where(w < n_full, n_chunks,
                         jnp.where(w == n_full, rem_chunks, 0))
        plsc.subcore_barrier()

        def remap(raw_ref, j, ring_ref, t):
            for k in range(CHUNK // LANES):
                sl = pl.ds(k * LANES, LANES)
                ring_ref[t, sl] = _gmap16(raw_ref[j, sl])

        def fire_gather(j, t):
            remap(src_v, j, gidx_v, t)
            pltpu.async_copy(hs_hbm.at[gidx_v.at[t]], rows_v.at[t], gsem)

        for t in range(nbuf - 1):
            @pl.when(t < n_my)
            def _prime(t=t):
                fire_gather(t, t)

        def body(j, _):
            for t in range(nbuf):
                jj = j * nbuf + t

                @pl.when(jnp.logical_and(jj > 0, jj < n_my))
                def _drain():
                    tp = (t - 1) % nbuf
                    pltpu.make_async_copy(
                        rows_v.at[tp], agg_sp.at[sidx_v.at[tp]], ssem
                    ).wait()

                @pl.when(jj + nbuf - 1 < n_my)
                def _prefetch():
                    fire_gather(jj + nbuf - 1, (t + nbuf - 1) % nbuf)

                @pl.when(jj < n_my)
                def _scat():
                    pltpu.make_async_copy(
                        hs_hbm.at[gidx_v.at[t]], rows_v.at[t], gsem
                    ).wait()
                    remap(dst_v, jj, sidx_v, t)
                    pltpu.async_copy(
                        rows_v.at[t], agg_sp.at[sidx_v.at[t]], ssem, add=True
                    )
            return _

        lax.fori_loop(0, (n_my + nbuf - 1) // nbuf, body, None)

        @pl.when(n_my > 0)
        def _last():
            tl = lax.rem(n_my - 1, nbuf)
            pltpu.make_async_copy(
                rows_v.at[tl], agg_sp.at[sidx_v.at[tl]], ssem
            ).wait()

        plsc.subcore_barrier()
        pltpu.sync_copy(
            agg_sp.at[pl.ds(s * stripe, stripe), :],
            agg_hbm.at[c, pl.ds(s * stripe, stripe), :],
        )

    return agg_kernel


def _dinv_cols(degp_ref, i, pk):
    sl = pl.ds(i * pk, pk)
    deg = degp_ref[0, sl, :] + degp_ref[1, sl, :] + 1.0
    dinv_t = lax.rsqrt(deg).T
    da = jnp.concatenate(
        [dinv_t[:, k:k + 1] for k in range(pk // 2)], axis=0)
    db = jnp.concatenate(
        [dinv_t[:, k:k + 1] for k in range(pk // 2, pk)], axis=0)
    return da, db


def _encode_block(pk, x_ref, w_ref, degp_ref, hs_ref):
    i = pl.program_id(0)
    da, db = _dinv_cols(degp_ref, i, pk)
    half = BLK // 2
    w = w_ref[...]
    ha = jnp.dot(x_ref[0:half, :] * da, w, preferred_element_type=jnp.float32)
    hb = jnp.dot(x_ref[half:BLK, :] * db, w, preferred_element_type=jnp.float32)
    hs_ref[...] = jnp.concatenate([ha, hb], axis=1)


def _decode_block(pk, aggp_ref, hs_ref, degp_ref, b_ref, wd_ref, out_ref):
    i = pl.program_id(0)
    da, db = _dinv_cols(degp_ref, i, pk)
    s128 = aggp_ref[0] + aggp_ref[1] + hs_ref[...]
    b = b_ref[...]
    wd = wd_ref[...]

    def half(sv, dv):
        z = jnp.maximum(sv * dv + b, 0.0)
        lg = lax.dot_general(z, wd, (((1,), (1,)), ((), ())),
                             preferred_element_type=jnp.float32)
        m = jnp.max(lg, axis=1, keepdims=True)
        e = jnp.exp(lg - m)
        return e / jnp.sum(e, axis=1, keepdims=True)

    oa = half(s128[:, 0:64], da)
    ob = half(s128[:, 64:128], db)
    out_ref[...] = jnp.concatenate([oa, ob], axis=0)


def kernel(x, edge_index, W_enc, b_enc, W_dec):
    n, in_ch = x.shape
    out_ch = W_enc.shape[1]
    e = edge_index.shape[1]

    ei = edge_index.astype(jnp.int32).reshape(2, e // CHUNK, CHUNK)
    n_chunks = -(-e // (NW * CHUNK))
    n_chunks = -(-n_chunks // 4) * 4
    epw = n_chunks * CHUNK
    n_full = e // epw
    rem_chunks = (e - n_full * epw) // CHUNK
    rows_pad = -(-n // BLK) * BLK
    stripe = rows_pad // NS

    degp = _make_deg_kernel(n_chunks, n_full, rem_chunks, rows_pad, stripe)(ei)
    degp3 = degp.reshape(NC, rows_pad // 128, 128)

    grid = rows_pad // BLK
    pk = BLK // 128
    pk2 = BLK * out_ch // 128
    hs128 = pl.pallas_call(
        functools.partial(_encode_block, pk),
        grid=(grid,),
        in_specs=[
            pl.BlockSpec((BLK, in_ch), lambda i: (i, 0)),
            pl.BlockSpec((in_ch, out_ch), lambda i: (0, 0)),
            pl.BlockSpec((NC, rows_pad // 128, 128), lambda i: (0, 0, 0)),
        ],
        out_specs=pl.BlockSpec((pk2, 128), lambda i: (i, 0)),
        out_shape=jax.ShapeDtypeStruct((rows_pad * out_ch // 128, 128),
                                       jnp.float32),
    )(x, W_enc, degp3)
    hs = hs128.reshape(rows_pad, out_ch)

    aggp = _make_agg_kernel(n_chunks, n_full, rem_chunks, rows_pad, stripe,
                            out_ch)(ei, hs)
    aggp128 = aggp.reshape(NC, rows_pad * out_ch // 128, 128)

    out = pl.pallas_call(
        functools.partial(_decode_block, pk),
        grid=(grid,),
        in_specs=[
            pl.BlockSpec((NC, pk2, 128), lambda i: (0, i, 0)),
            pl.BlockSpec((pk2, 128), lambda i: (i, 0)),
            pl.BlockSpec((NC, rows_pad // 128, 128), lambda i: (0, 0, 0)),
            pl.BlockSpec((1, out_ch), lambda i: (0, 0)),
            pl.BlockSpec((in_ch, out_ch), lambda i: (0, 0)),
        ],
        out_specs=pl.BlockSpec((BLK, in_ch), lambda i: (i, 0)),
        out_shape=jax.ShapeDtypeStruct((n, in_ch), jnp.float32),
    )(aggp128, hs128, degp3, b_enc.reshape(1, out_ch), W_dec)

    return out


# speedup vs baseline: 1.3037x; 1.0424x over previous
import functools

import jax
import jax.numpy as jnp
from jax import lax
from jax.experimental import pallas as pl
from jax.experimental.pallas import tpu as pltpu
from jax.experimental.pallas import tpu_sc as plsc

NC = 2
NS = 16
NW = NC * NS
LANES = 16
CHUNK = 128
BLK = 1024


def _sc_mesh():
    return plsc.VectorSubcoreMesh(
        core_axis_name="c", subcore_axis_name="s", num_cores=NC, num_subcores=NS
    )


_SC_PARAMS = pltpu.CompilerParams(use_tc_tiling_on_sc=False)


def _zeros16():
    return jnp.zeros((LANES,), jnp.float32)


def _gmap16(v):
    u = lax.bitwise_and(v, BLK - 1)
    p = lax.shift_right_logical(u, (BLK // 2).bit_length() - 1)
    r = lax.bitwise_and(u, BLK // 2 - 1)
    return (v - u) + 2 * r + p


def _make_deg_kernel(n_chunks, n_full, rem_chunks, rows_pad, stripe):
    epw = n_chunks * CHUNK

    @functools.partial(
        pl.kernel,
        mesh=_sc_mesh(),
        out_type=jax.ShapeDtypeStruct((NC, rows_pad), jnp.float32),
        compiler_params=_SC_PARAMS,
        scratch_types=[
            pltpu.VMEM_SHARED((rows_pad,), jnp.float32),
            pltpu.VMEM((n_chunks, CHUNK), jnp.int32),
            pltpu.VMEM((CHUNK,), jnp.float32),
            pltpu.VMEM((stripe,), jnp.float32),
            pltpu.SemaphoreType.DMA,
        ],
    )
    def deg_kernel(ei_hbm, degp_hbm, deg_sp, idx_v, ones_v, zbuf_v, dsem):
        c = lax.axis_index("c")
        s = lax.axis_index("s")
        w = c * NS + s
        for i in range(stripe // LANES):
            zbuf_v[pl.ds(i * LANES, LANES)] = _zeros16()
        for i in range(CHUNK // LANES):
            ones_v[pl.ds(i * LANES, LANES)] = jnp.full((LANES,), 1.0, jnp.float32)
        pltpu.sync_copy(zbuf_v, deg_sp.at[pl.ds(s * stripe, stripe)])

        @pl.when(w < n_full)
        def _full():
            pltpu.sync_copy(ei_hbm.at[1, pl.ds(w * n_chunks, n_chunks), :],
                            idx_v)

        if rem_chunks:
            @pl.when(w == n_full)
            def _rem():
                pltpu.sync_copy(
                    ei_hbm.at[1, pl.ds(w * n_chunks, rem_chunks), :],
                    idx_v.at[pl.ds(0, rem_chunks), :])

        n_my = jnp.where(w < n_full, n_chunks,
                         jnp.where(w == n_full, rem_chunks, 0))
        plsc.subcore_barrier()

        def body(g, _):
            for t in range(4):
                pltpu.async_copy(ones_v, deg_sp.at[idx_v.at[g * 4 + t]],
                                 dsem, add=True)

            @pl.when(g > 0)
            def _drain():
                for _t in range(4):
                    pltpu.make_async_copy(
                        ones_v, deg_sp.at[idx_v.at[0]], dsem).wait()
            return _

        n_groups = lax.div(n_my, 4)
        lax.fori_loop(0, n_groups, body, None)

        @pl.when(n_my > 0)
        def _drain_last():
            for _t in range(4):
                pltpu.make_async_copy(
                    ones_v, deg_sp.at[idx_v.at[0]], dsem).wait()

        plsc.subcore_barrier()
        pltpu.sync_copy(
            deg_sp.at[pl.ds(s * stripe, stripe)],
            degp_hbm.at[c, pl.ds(s * stripe, stripe)],
        )

    return deg_kernel


def _make_agg_kernel(n_chunks, n_full, rem_chunks, rows_pad, stripe, out_ch):
    epw = n_chunks * CHUNK
    nbuf = 4

    @functools.partial(
        pl.kernel,
        mesh=_sc_mesh(),
        out_type=jax.ShapeDtypeStruct((NC, rows_pad, out_ch), jnp.float32),
        compiler_params=_SC_PARAMS,
        scratch_types=[
            pltpu.VMEM_SHARED((rows_pad, out_ch), jnp.float32),
            pltpu.VMEM((n_chunks, CHUNK), jnp.int32),
            pltpu.VMEM((n_chunks, CHUNK), jnp.int32),
            pltpu.VMEM((nbuf, CHUNK), jnp.int32),
            pltpu.VMEM((nbuf, CHUNK), jnp.int32),
            pltpu.VMEM((nbuf, CHUNK, out_ch), jnp.float32),
            pltpu.VMEM((LANES, out_ch), jnp.float32),
            pltpu.SemaphoreType.DMA,
            pltpu.SemaphoreType.DMA,
        ],
    )
    def agg_kernel(ei_hbm, hs_hbm, agg_hbm,
                   agg_sp, src_v, dst_v, gidx_v, sidx_v, rows_v, zbuf,
                   gsem, ssem):
        c = lax.axis_index("c")
        s = lax.axis_index("s")
        w = c * NS + s
        for r in range(LANES):
            for q in range(out_ch // LANES):
                zbuf[r, pl.ds(q * LANES, LANES)] = _zeros16()

        def zbody(k, _):
            pltpu.sync_copy(
                zbuf, agg_sp.at[pl.ds(s * stripe + k * LANES, LANES), :]
            )
            return _

        lax.fori_loop(0, stripe // LANES, zbody, None)

        @pl.when(w < n_full)
        def _full():
            pltpu.sync_copy(ei_hbm.at[0, pl.ds(w * n_chunks, n_chunks), :],
                            src_v)
            pltpu.sync_copy(ei_hbm.at[1, pl.ds(w * n_chunks, n_chunks), :],
                            dst_v)

        if rem_chunks:
            @pl.when(w == n_full)
            def _rem():
                pltpu.sync_copy(ei_hbm.at[0, pl.ds(w * n_chunks, rem_chunks), :],
                                src_v.at[pl.ds(0, rem_chunks), :])
                pltpu.sync_copy(ei_hbm.at[1, pl.ds(w * n_chunks, rem_chunks), :],
                                dst_v.at[pl.ds(0, rem_chunks), :])

        n_my = jnp.where(w < n_full, n_chunks,
                         jnp.where(w == n_full, rem_chunks, 0))
        plsc.subcore_barrier()

        def remap(raw_ref, j, ring_ref, t):
            for k in range(CHUNK // LANES):
                sl = pl.ds(k * LANES, LANES)
                ring_ref[t, sl] = _gmap16(raw_ref[j, sl])

        def fire_gather(j, t):
            remap(src_v, j, gidx_v, t)
            pltpu.async_copy(hs_hbm.at[gidx_v.at[t]], rows_v.at[t], gsem)

        for t in range(nbuf - 1):
            @pl.when(t < n_my)
            def _prime(t=t):
                fire_gather(t, t)

        def body(j, _):
            for t in range(nbuf):
                jj = j * nbuf + t

                @pl.when(jnp.logical_and(jj > 0, jj < n_my))
                def _drain():
                    tp = (t - 1) % nbuf
                    pltpu.make_async_copy(
                        rows_v.at[tp], agg_sp.at[sidx_v.at[tp]], ssem
                    ).wait()

                @pl.when(jj + nbuf - 1 < n_my)
                def _prefetch():
                    fire_gather(jj + nbuf - 1, (t + nbuf - 1) % nbuf)

                @pl.when(jj < n_my)
                def _scat():
                    pltpu.make_async_copy(
                        hs_hbm.at[gidx_v.at[t]], rows_v.at[t], gsem
                    ).wait()
                    remap(dst_v, jj, sidx_v, t)
                    pltpu.async_copy(
                        rows_v.at[t], agg_sp.at[sidx_v.at[t]], ssem, add=True
                    )
            return _

        lax.fori_loop(0, (n_my + nbuf - 1) // nbuf, body, None)

        @pl.when(n_my > 0)
        def _last():
            tl = lax.rem(n_my - 1, nbuf)
            pltpu.make_async_copy(
                rows_v.at[tl], agg_sp.at[sidx_v.at[tl]], ssem
            ).wait()

        plsc.subcore_barrier()
        pltpu.sync_copy(
            agg_sp.at[pl.ds(s * stripe, stripe), :],
            agg_hbm.at[c, pl.ds(s * stripe, stripe), :],
        )

    return agg_kernel


def _dinv_cols(degp_ref, i, pk):
    sl = pl.ds(i * pk, pk)
    deg = degp_ref[0, sl, :] + degp_ref[1, sl, :] + 1.0
    dinv_t = lax.rsqrt(deg).T
    da = jnp.concatenate(
        [dinv_t[:, k:k + 1] for k in range(pk // 2)], axis=0)
    db = jnp.concatenate(
        [dinv_t[:, k:k + 1] for k in range(pk // 2, pk)], axis=0)
    return da, db


def _encode_block(pk, x_ref, w_ref, degp_ref, hs_ref):
    i = pl.program_id(0)
    da, db = _dinv_cols(degp_ref, i, pk)
    half = BLK // 2
    w = w_ref[...]
    ha = jnp.dot(x_ref[0:half, :] * da, w, preferred_element_type=jnp.float32)
    hb = jnp.dot(x_ref[half:BLK, :] * db, w, preferred_element_type=jnp.float32)
    hs_ref[...] = jnp.concatenate([ha, hb], axis=1)


def _decode_block(pk, aggp_ref, hs_ref, degp_ref, b_ref, wd_ref, out_ref):
    i = pl.program_id(0)
    da, db = _dinv_cols(degp_ref, i, pk)
    s128 = aggp_ref[0] + aggp_ref[1] + hs_ref[...]
    b = b_ref[...]
    wd = wd_ref[...]

    def half(sv, dv):
        z = jnp.maximum(sv * dv + b, 0.0)
        lg = lax.dot_general(z, wd, (((1,), (1,)), ((), ())),
                             preferred_element_type=jnp.float32)
        m = jnp.max(lg, axis=1, keepdims=True)
        e = jnp.exp(lg - m)
        return e / jnp.sum(e, axis=1, keepdims=True)

    oa = half(s128[:, 0:64], da)
    ob = half(s128[:, 64:128], db)
    out_ref[...] = jnp.concatenate([oa, ob], axis=0)


def kernel(x, edge_index, W_enc, b_enc, W_dec):
    n, in_ch = x.shape
    out_ch = W_enc.shape[1]
    e = edge_index.shape[1]

    ei = edge_index.astype(jnp.int32).reshape(2, e // CHUNK, CHUNK)
    n_chunks = -(-e // (NW * CHUNK))
    n_chunks = -(-n_chunks // 4) * 4
    epw = n_chunks * CHUNK
    n_full = e // epw
    rem_chunks = (e - n_full * epw) // CHUNK
    rows_pad = -(-n // BLK) * BLK
    stripe = rows_pad // NS

    degp = _make_deg_kernel(n_chunks, n_full, rem_chunks, rows_pad, stripe)(ei)
    degp3 = degp.reshape(NC, rows_pad // 128, 128)

    grid = rows_pad // BLK
    pk = BLK // 128
    pk2 = BLK * out_ch // 128
    hs128 = pl.pallas_call(
        functools.partial(_encode_block, pk),
        grid=(grid,),
        in_specs=[
            pl.BlockSpec((BLK, in_ch), lambda i: (i, 0)),
            pl.BlockSpec((in_ch, out_ch), lambda i: (0, 0)),
            pl.BlockSpec((NC, rows_pad // 128, 128), lambda i: (0, 0, 0)),
        ],
        out_specs=pl.BlockSpec((pk2, 128), lambda i: (i, 0)),
        out_shape=jax.ShapeDtypeStruct((rows_pad * out_ch // 128, 128),
                                       jnp.float32),
    )(x, W_enc, degp3)
    hs = hs128.reshape(rows_pad, out_ch)

    aggp = _make_agg_kernel(n_chunks, n_full, rem_chunks, rows_pad, stripe,
                            out_ch)(ei, hs)
    aggp128 = aggp.reshape(NC, rows_pad * out_ch // 128, 128)

    out = pl.pallas_call(
        functools.partial(_decode_block, pk),
        grid=(grid,),
        in_specs=[
            pl.BlockSpec((NC, pk2, 128), lambda i: (0, i, 0)),
            pl.BlockSpec((pk2, 128), lambda i: (i, 0)),
            pl.BlockSpec((NC, rows_pad // 128, 128), lambda i: (0, 0, 0)),
            pl.BlockSpec((1, out_ch), lambda i: (0, 0)),
            pl.BlockSpec((in_ch, out_ch), lambda i: (0, 0)),
        ],
        out_specs=pl.BlockSpec((BLK, in_ch), lambda i: (i, 0)),
        out_shape=jax.ShapeDtypeStruct((n, in_ch), jnp.float32),
    )(aggp128, hs128, degp3, b_enc.reshape(1, out_ch), W_dec)

    return out
